# baseline (speedup 1.0000x reference)
import jax
import jax.numpy as jnp
from jax.experimental import pallas as pl
from jax.experimental.pallas import tpu as pltpu

_B, _N, _C, _H = 4, 2048, 256, 512
_TR = 256
_P1 = _B
_P2 = _N // _TR


def _fused_kernel(node_ref, adj_ref, g1_ref, b1_ref, g2_ref, b2_ref,
                  w11_ref, bb11_ref, w12_ref, bb12_ref, w21_ref, bb21_ref,
                  w22_ref, bb22_ref, out_ref, xt_s):
    i = pl.program_id(0)

    @pl.when(i < _P1)
    def _phase1():
        x = node_ref[...]
        m = jnp.mean(x, axis=-1, keepdims=True)
        v = jnp.mean((x - m) ** 2, axis=-1, keepdims=True)
        xn = (x - m) * jax.lax.rsqrt(v + 1e-5) * g1_ref[...] + b1_ref[...]
        t = (jnp.dot(xn, w11_ref[...], preferred_element_type=jnp.float32)
             + bb11_ref[...])
        h = 0.5 * t * (1.0 + jax.lax.erf(t * 0.7071067811865476))
        x1 = (jnp.dot(h, w12_ref[...], preferred_element_type=jnp.float32)
              + bb12_ref[...] + x)
        xt_s[:, pl.ds(i * _C, _C)] = x1

    @pl.when(i >= _P1)
    def _phase2():
        jj = (i - _P1) * _TR
        agg_all = jnp.dot(adj_ref[...], xt_s[...],
                          preferred_element_type=jnp.float32)
        g2 = g2_ref[...]
        b2 = b2_ref[...]
        w21 = w21_ref[...]
        w22 = w22_ref[...]
        for b in range(_B):
            agg = agg_all[:, b * _C:(b + 1) * _C]
            x1 = xt_s[pl.ds(jj, _TR), b * _C:(b + 1) * _C]
            s = (jnp.sum(x1, axis=-1, keepdims=True)
                 + jnp.sum(agg, axis=-1, keepdims=True))
            m = s / (2 * _C)
            q = (jnp.sum((x1 - m) ** 2, axis=-1, keepdims=True)
                 + jnp.sum((agg - m) ** 2, axis=-1, keepdims=True))
            rs = jax.lax.rsqrt(q / (2 * _C) + 1e-5)
            xa = (x1 - m) * rs * g2[:, :_C] + b2[:, :_C]
            xb = (agg - m) * rs * g2[:, _C:] + b2[:, _C:]
            t = (jnp.dot(xa, w21[:_C], preferred_element_type=jnp.float32)
                 + jnp.dot(xb, w21[_C:], preferred_element_type=jnp.float32)
                 + bb21_ref[...])
            h = 0.5 * t * (1.0 + jax.lax.erf(t * 0.7071067811865476))
            out_ref[b] = (jnp.dot(h, w22_ref[...],
                                  preferred_element_type=jnp.float32)
                          + bb22_ref[...] + x1)


def _node_map(i):
    return (jnp.minimum(i, _P1 - 1), 0)


def _j_map(i):
    return (jnp.maximum(i - _P1, 0), 0)


@jax.jit
def kernel(node, edge, adj, g1, b1, g2, b2, w11, bb11, w12, bb12, w21, bb21,
           w22, bb22):
    B, N, C = node.shape
    H = w11.shape[1]
    flat = node.reshape(B * N, C)
    rep = lambda i: (0, 0)

    out = pl.pallas_call(
        _fused_kernel,
        grid=(_P1 + _P2,),
        in_specs=[
            pl.BlockSpec((N, C), _node_map),
            pl.BlockSpec((_TR, N), _j_map),
            pl.BlockSpec((1, C), rep),
            pl.BlockSpec((1, C), rep),
            pl.BlockSpec((1, 2 * C), rep),
            pl.BlockSpec((1, 2 * C), rep),
            pl.BlockSpec((C, H), rep),
            pl.BlockSpec((1, H), rep),
            pl.BlockSpec((H, C), rep),
            pl.BlockSpec((1, C), rep),
            pl.BlockSpec((2 * C, H), rep),
            pl.BlockSpec((1, H), rep),
            pl.BlockSpec((H, C), rep),
            pl.BlockSpec((1, C), rep),
        ],
        out_specs=pl.BlockSpec((B, _TR, C), lambda i: (0, jnp.maximum(i - _P1, 0), 0)),
        out_shape=jax.ShapeDtypeStruct((B, N, C), jnp.float32),
        scratch_shapes=[
            pltpu.VMEM((N, B * C), jnp.float32),
        ],
    )(flat, adj, g1.reshape(1, C), b1.reshape(1, C), g2.reshape(1, 2 * C),
      b2.reshape(1, 2 * C), w11, bb11.reshape(1, H), w12, bb12.reshape(1, C),
      w21, bb21.reshape(1, H), w22, bb22.reshape(1, C))

    return (out, edge)


# speedup vs baseline: 1.0507x; 1.0507x over previous
import jax
import jax.numpy as jnp
from jax.experimental import pallas as pl
from jax.experimental.pallas import tpu as pltpu

_B, _N, _C, _H = 4, 2048, 256, 512
_TR = 512
_P1 = _B
_P2 = _N // _TR


def _fused_kernel(node_ref, adj_ref, g1_ref, b1_ref, g2_ref, b2_ref,
                  w11_ref, bb11_ref, w12_ref, bb12_ref, w21_ref, bb21_ref,
                  w22_ref, bb22_ref, out_ref, xt_s):
    i = pl.program_id(0)

    @pl.when(i < _P1)
    def _phase1():
        x = node_ref[...]
        m = jnp.mean(x, axis=-1, keepdims=True)
        v = jnp.mean((x - m) ** 2, axis=-1, keepdims=True)
        xn = (x - m) * jax.lax.rsqrt(v + 1e-5) * g1_ref[...] + b1_ref[...]
        t = (jnp.dot(xn, w11_ref[...], preferred_element_type=jnp.float32)
             + bb11_ref[...])
        h = 0.5 * t * (1.0 + jax.lax.erf(t * 0.7071067811865476))
        x1 = (jnp.dot(h, w12_ref[...], preferred_element_type=jnp.float32)
              + bb12_ref[...] + x)
        xt_s[:, pl.ds(i * _C, _C)] = x1

    @pl.when(i >= _P1)
    def _phase2():
        jj = (i - _P1) * _TR
        agg_all = jnp.dot(adj_ref[...], xt_s[...],
                          preferred_element_type=jnp.float32)
        g2 = g2_ref[...]
        b2 = b2_ref[...]
        w21 = w21_ref[...]
        w22 = w22_ref[...]
        for b in range(_B):
            agg = agg_all[:, b * _C:(b + 1) * _C]
            x1 = xt_s[pl.ds(jj, _TR), b * _C:(b + 1) * _C]
            s = (jnp.sum(x1, axis=-1, keepdims=True)
                 + jnp.sum(agg, axis=-1, keepdims=True))
            m = s / (2 * _C)
            q = (jnp.sum((x1 - m) ** 2, axis=-1, keepdims=True)
                 + jnp.sum((agg - m) ** 2, axis=-1, keepdims=True))
            rs = jax.lax.rsqrt(q / (2 * _C) + 1e-5)
            xa = (x1 - m) * rs * g2[:, :_C] + b2[:, :_C]
            xb = (agg - m) * rs * g2[:, _C:] + b2[:, _C:]
            t = (jnp.dot(xa, w21[:_C], preferred_element_type=jnp.float32)
                 + jnp.dot(xb, w21[_C:], preferred_element_type=jnp.float32)
                 + bb21_ref[...])
            h = 0.5 * t * (1.0 + jax.lax.erf(t * 0.7071067811865476))
            out_ref[b] = (jnp.dot(h, w22_ref[...],
                                  preferred_element_type=jnp.float32)
                          + bb22_ref[...] + x1)


def _node_map(i):
    return (jnp.minimum(i, _P1 - 1), 0)


def _j_map(i):
    return (jnp.maximum(i - _P1, 0), 0)


@jax.jit
def kernel(node, edge, adj, g1, b1, g2, b2, w11, bb11, w12, bb12, w21, bb21,
           w22, bb22):
    B, N, C = node.shape
    H = w11.shape[1]
    flat = node.reshape(B * N, C)
    rep = lambda i: (0, 0)

    out = pl.pallas_call(
        _fused_kernel,
        grid=(_P1 + _P2,),
        in_specs=[
            pl.BlockSpec((N, C), _node_map),
            pl.BlockSpec((_TR, N), _j_map),
            pl.BlockSpec((1, C), rep),
            pl.BlockSpec((1, C), rep),
            pl.BlockSpec((1, 2 * C), rep),
            pl.BlockSpec((1, 2 * C), rep),
            pl.BlockSpec((C, H), rep),
            pl.BlockSpec((1, H), rep),
            pl.BlockSpec((H, C), rep),
            pl.BlockSpec((1, C), rep),
            pl.BlockSpec((2 * C, H), rep),
            pl.BlockSpec((1, H), rep),
            pl.BlockSpec((H, C), rep),
            pl.BlockSpec((1, C), rep),
        ],
        out_specs=pl.BlockSpec((B, _TR, C), lambda i: (0, jnp.maximum(i - _P1, 0), 0)),
        out_shape=jax.ShapeDtypeStruct((B, N, C), jnp.float32),
        scratch_shapes=[
            pltpu.VMEM((N, B * C), jnp.float32),
        ],
    )(flat, adj, g1.reshape(1, C), b1.reshape(1, C), g2.reshape(1, 2 * C),
      b2.reshape(1, 2 * C), w11, bb11.reshape(1, H), w12, bb12.reshape(1, C),
      w21, bb21.reshape(1, H), w22, bb22.reshape(1, C))

    return (out, edge)
